# baseline (speedup 1.0000x reference)
import functools
import jax
import jax.numpy as jnp
from jax import lax
from jax.experimental import pallas as pl
from jax.experimental.pallas import tpu as pltpu
from jax.experimental.pallas import tpu_sc as plsc

_EPS = 0.1
_V = 100000
_N = 2048
_FILL = _EPS / (_V - 2)

_R = 256
_C = 2048
_NC = (_V + _C - 1) // _C
_L = 128

_NCORES = 2
_NSUB = 16
_NW = _NCORES * _NSUB
_BPW = _N // _NW


def _stats_kernel(pred_ref, m_ref, s_ref, tot_ref):
    j = pl.program_id(1)

    @pl.when(j == 0)
    def _init():
        m_ref[...] = jnp.full((_R, _L), -1e30, jnp.float32)
        s_ref[...] = jnp.zeros((_R, _L), jnp.float32)
        tot_ref[...] = jnp.zeros((_R, _L), jnp.float32)

    def _update(x3):
        cmax = jnp.max(x3, axis=1)
        m_old = m_ref[...]
        m_new = jnp.maximum(m_old, cmax)
        e = jnp.exp(x3 - m_new[:, None, :])
        s_ref[...] = s_ref[...] * jnp.exp(m_old - m_new) + jnp.sum(e, axis=1)
        tot_ref[...] += jnp.sum(x3, axis=1)
        m_ref[...] = m_new

    @pl.when(j < _NC - 1)
    def _full():
        _update(pred_ref[...].reshape(_R, _C // _L, _L))

    @pl.when(j == _NC - 1)
    def _ragged():
        x = pred_ref[...]
        cols = lax.broadcasted_iota(jnp.int32, (_R, _C), 1) + j * _C
        valid = cols < _V
        xm = jnp.where(valid, x, -1e30).reshape(_R, _C // _L, _L)
        xs = jnp.where(valid, x, 0.0).reshape(_R, _C // _L, _L)
        cmax = jnp.max(xm, axis=1)
        m_old = m_ref[...]
        m_new = jnp.maximum(m_old, cmax)
        e = jnp.exp(xm - m_new[:, None, :])
        s_ref[...] = s_ref[...] * jnp.exp(m_old - m_new) + jnp.sum(e, axis=1)
        tot_ref[...] += jnp.sum(xs, axis=1)
        m_ref[...] = m_new


@functools.cache
def _make_sc_gather():
    mesh = plsc.VectorSubcoreMesh(core_axis_name="c", subcore_axis_name="s")

    @functools.partial(
        pl.kernel,
        mesh=mesh,
        out_type=jax.ShapeDtypeStruct((_N,), jnp.float32),
        scratch_types=[
            pltpu.VMEM((_BPW,), jnp.int32),
            pltpu.VMEM((_BPW,), jnp.int32),
            pltpu.VMEM((_BPW,), jnp.float32),
            pltpu.SemaphoreType.DMA,
        ],
    )
    def _sc_gather(pred_flat_hbm, tgt_hbm, out_hbm, tgt_v, idx_v, val_v, sem):
        wid = lax.axis_index("s") * _NCORES + lax.axis_index("c")
        base = wid * _BPW
        pltpu.sync_copy(tgt_hbm.at[pl.ds(base, _BPW)], tgt_v)
        for k in range(_BPW // 16):
            rows = lax.iota(jnp.int32, 16) + (base + k * 16)
            idx_v[pl.ds(k * 16, 16)] = rows * _V + tgt_v[pl.ds(k * 16, 16)]
        pltpu.async_copy(pred_flat_hbm.at[idx_v], val_v, sem).wait()
        pltpu.sync_copy(val_v, out_hbm.at[pl.ds(base, _BPW)])

    return _sc_gather


def _combine_kernel(m_ref, s_ref, tot_ref, tv_ref, tgt_ref, out_ref):
    m_l = m_ref[...]
    big_m = jnp.max(m_l, axis=1, keepdims=True)
    s = jnp.sum(s_ref[...] * jnp.exp(m_l - big_m), axis=1, keepdims=True)
    tot = jnp.sum(tot_ref[...], axis=1, keepdims=True)
    logz = big_m + jnp.log(s)
    s_row = tot - _V * logz
    logp_t = tv_ref[...] - logz
    loss = -(_FILL * s_row + (1.0 - _EPS - _FILL) * logp_t)
    mask = tgt_ref[...] != 0
    loss_sum = jnp.sum(jnp.where(mask, loss, 0.0), keepdims=True).reshape(1, 1)
    cnt = jnp.sum(mask.astype(jnp.float32), keepdims=True).reshape(1, 1)
    out_ref[...] = jnp.where(cnt > 0, loss_sum / jnp.maximum(cnt, 1.0), 0.0)


def kernel(pred, target):
    tv = _make_sc_gather()(pred.reshape(_N * _V), target)

    m, s, tot = pl.pallas_call(
        _stats_kernel,
        grid=(_N // _R, _NC),
        in_specs=[pl.BlockSpec((_R, _C), lambda i, j: (i, j))],
        out_specs=[pl.BlockSpec((_R, _L), lambda i, j: (i, 0))] * 3,
        out_shape=[jax.ShapeDtypeStruct((_N, _L), jnp.float32)] * 3,
    )(pred)

    out = pl.pallas_call(
        _combine_kernel,
        out_shape=jax.ShapeDtypeStruct((1, 1), jnp.float32),
    )(m, s, tot, tv.reshape(_N, 1), target.reshape(_N, 1))
    return out[0, 0]


# speedup vs baseline: 2.1640x; 2.1640x over previous
import functools
import jax
import jax.numpy as jnp
from jax import lax
from jax.experimental import pallas as pl
from jax.experimental.pallas import tpu as pltpu
from jax.experimental.pallas import tpu_sc as plsc

_EPS = 0.1
_V = 100000
_N = 2048
_FILL = _EPS / (_V - 2)

_VR = 16


def _stats_kernel(pred_ref, tgt_ref, m_ref, s_ref, tot_ref, tv_ref):
    x = pred_ref[...]
    m = jnp.max(x, axis=1, keepdims=True)
    tot = jnp.sum(x, axis=1, keepdims=True)
    s = jnp.sum(jnp.exp(x - m), axis=1, keepdims=True)
    cols = lax.broadcasted_iota(jnp.int32, (_VR, _V), 1)
    tv = jnp.sum(jnp.where(cols == tgt_ref[...], x, 0.0), axis=1,
                 keepdims=True)
    m_ref[...] = m
    s_ref[...] = s
    tot_ref[...] = tot
    tv_ref[...] = tv


def _combine_kernel(m_ref, s_ref, tot_ref, tv_ref, tgt_ref, out_ref):
    logz = m_ref[...] + jnp.log(s_ref[...])
    s_row = tot_ref[...] - _V * logz
    logp_t = tv_ref[...] - logz
    loss = -(_FILL * s_row + (1.0 - _EPS - _FILL) * logp_t)
    mask = tgt_ref[...] != 0
    loss_sum = jnp.sum(jnp.where(mask, loss, 0.0), keepdims=True).reshape(1, 1)
    cnt = jnp.sum(mask.astype(jnp.float32), keepdims=True).reshape(1, 1)
    out_ref[...] = jnp.where(cnt > 0, loss_sum / jnp.maximum(cnt, 1.0), 0.0)


def kernel(pred, target):
    m, s, tot, tv = pl.pallas_call(
        _stats_kernel,
        grid=(_N // _VR,),
        in_specs=[
            pl.BlockSpec((_VR, _V), lambda i: (i, 0)),
            pl.BlockSpec((_VR, 1), lambda i: (i, 0)),
        ],
        out_specs=[pl.BlockSpec((_VR, 1), lambda i: (i, 0))] * 4,
        out_shape=[jax.ShapeDtypeStruct((_N, 1), jnp.float32)] * 4,
    )(pred, target.reshape(_N, 1))

    out = pl.pallas_call(
        _combine_kernel,
        out_shape=jax.ShapeDtypeStruct((1, 1), jnp.float32),
    )(m.reshape(16, 128), s.reshape(16, 128), tot.reshape(16, 128),
      tv.reshape(16, 128), target.reshape(16, 128))
    return out[0, 0]
